# Initial kernel scaffold; baseline (speedup 1.0000x reference)
import jax
import jax.numpy as jnp
from jax.experimental import pallas as pl


def kernel(logprobs, beam_seq, beam_seq_logprobs, beam_logprobs_sum, state, t):
    raise NotImplementedError("write your pallas kernel here")


# speedup vs baseline: 38.9912x; 38.9912x over previous
import functools

import jax
import jax.numpy as jnp
from jax.experimental import pallas as pl

UNK = 3
NEG = float("-inf")


def _beam_step_kernel(lp_ref, seq_ref, seqlp_ref, blps_ref, state_ref, t_ref,
                      seq_out, seqlp_out, sums_out, state_out, *, beam, V, L):
    x = lp_ref[0]
    lane = jax.lax.broadcasted_iota(jnp.int32, (beam, V), 1)
    x = jnp.where(lane == UNK, x - 1000.0, x)

    vals_cols = []
    idx_cols = []
    for _ in range(beam):
        m = jnp.max(x, axis=1, keepdims=True)
        am = jnp.min(jnp.where(x >= m, lane, V), axis=1, keepdims=True)
        vals_cols.append(m)
        idx_cols.append(am)
        x = jnp.where(lane == am, NEG, x)
    top_vals = jnp.concatenate(vals_cols, axis=1)
    top_idx = jnp.concatenate(idx_cols, axis=1)

    row_bb = jax.lax.broadcasted_iota(jnp.int32, (beam, beam), 0)
    lane_bb = jax.lax.broadcasted_iota(jnp.int32, (beam, beam), 1)
    bl = blps_ref[0]
    bl_col = jnp.sum(jnp.where(lane_bb == row_bb,
                               jnp.broadcast_to(bl, (beam, beam)), 0.0),
                     axis=1, keepdims=True)
    cand = top_vals + bl_col
    pos = row_bb * beam + lane_bb

    sums_row = jnp.zeros((1, beam), jnp.float32)
    tok_col = jnp.zeros((beam, 1), jnp.int32)
    slp_col = jnp.zeros((beam, 1), jnp.float32)
    src_col = jnp.zeros((beam, 1), jnp.int32)
    lane_1b = jax.lax.broadcasted_iota(jnp.int32, (1, beam), 1)
    row_b1 = jax.lax.broadcasted_iota(jnp.int32, (beam, 1), 0)
    for i in range(beam):
        m = jnp.max(cand)
        p = jnp.min(jnp.where(cand >= m, pos, beam * beam))
        tok = jnp.sum(jnp.where(pos == p, top_idx, 0))
        slp = jnp.sum(jnp.where(pos == p, top_vals, 0.0))
        cand = jnp.where(pos == p, NEG, cand)
        sums_row = jnp.where(lane_1b == i, m, sums_row)
        tok_col = jnp.where(row_b1 == i, tok, tok_col)
        slp_col = jnp.where(row_b1 == i, slp, slp_col)
        src_col = jnp.where(row_b1 == i, p // beam, src_col)

    seq = seq_ref[0]
    seqlp = seqlp_ref[0]
    new_seq = jnp.zeros((beam, L), jnp.int32)
    new_seqlp = jnp.zeros((beam, L), jnp.float32)
    for r in range(beam):
        sel = src_col == r
        new_seq = jnp.where(sel, seq[r:r + 1, :], new_seq)
        new_seqlp = jnp.where(sel, seqlp[r:r + 1, :], new_seqlp)
    col_L = jax.lax.broadcasted_iota(jnp.int32, (beam, L), 1)
    t = t_ref[0, 0]
    new_seq = jnp.where(col_L == t, tok_col, new_seq)
    new_seqlp = jnp.where(col_L == t, slp_col, new_seqlp)

    seq_out[0] = new_seq
    seqlp_out[0] = new_seqlp
    sums_out[0] = sums_row

    layers = state_ref.shape[0]
    H = state_ref.shape[3]
    for layer in range(layers):
        s = state_ref[layer, 0]
        ns = jnp.zeros((beam, H), jnp.float32)
        for r in range(beam):
            ns = jnp.where(src_col == r, s[r:r + 1, :], ns)
        state_out[layer, 0] = ns


def kernel(logprobs, beam_seq, beam_seq_logprobs, beam_logprobs_sum, state, t):
    B, beam, V = logprobs.shape
    L = beam_seq.shape[2]
    layers, _, _, H = state.shape
    blps3 = beam_logprobs_sum.reshape(B, 1, beam)
    t_arr = jnp.asarray(t, jnp.int32).reshape(1, 1)

    body = functools.partial(_beam_step_kernel, beam=beam, V=V, L=L)
    grid = (B,)
    out = pl.pallas_call(
        body,
        grid=grid,
        in_specs=[
            pl.BlockSpec((1, beam, V), lambda b: (b, 0, 0)),
            pl.BlockSpec((1, beam, L), lambda b: (b, 0, 0)),
            pl.BlockSpec((1, beam, L), lambda b: (b, 0, 0)),
            pl.BlockSpec((1, 1, beam), lambda b: (b, 0, 0)),
            pl.BlockSpec((layers, 1, beam, H), lambda b: (0, b, 0, 0)),
            pl.BlockSpec((1, 1), lambda b: (0, 0)),
        ],
        out_specs=[
            pl.BlockSpec((1, beam, L), lambda b: (b, 0, 0)),
            pl.BlockSpec((1, beam, L), lambda b: (b, 0, 0)),
            pl.BlockSpec((1, 1, beam), lambda b: (b, 0, 0)),
            pl.BlockSpec((layers, 1, beam, H), lambda b: (0, b, 0, 0)),
        ],
        out_shape=[
            jax.ShapeDtypeStruct((B, beam, L), jnp.int32),
            jax.ShapeDtypeStruct((B, beam, L), jnp.float32),
            jax.ShapeDtypeStruct((B, 1, beam), jnp.float32),
            jax.ShapeDtypeStruct((layers, B, beam, H), jnp.float32),
        ],
    )(logprobs, beam_seq, beam_seq_logprobs, blps3, state, t_arr)
    new_seq, new_seqlp, sums3, new_state = out
    return (new_seq, new_seqlp, sums3.reshape(B, beam), new_state)


# speedup vs baseline: 41.0534x; 1.0529x over previous
import jax
import jax.numpy as jnp
from jax.experimental import pallas as pl

UNK = 3
NEG = float("-inf")


def _scan_kernel(x_ref, vals_ref, idx_ref):
    x = x_ref[0]
    R, V = x.shape
    lane = jax.lax.broadcasted_iota(jnp.int32, (R, V), 1)
    x = jnp.where(lane == UNK, x - 1000.0, x)
    vals_cols = []
    idx_cols = []
    for _ in range(5):
        m = jnp.max(x, axis=1, keepdims=True)
        am = jnp.min(jnp.where(x >= m, lane, V), axis=1, keepdims=True)
        vals_cols.append(m)
        idx_cols.append(am)
        x = jnp.where(lane == am, NEG, x)
    vals_ref[0] = jnp.concatenate(vals_cols, axis=1)
    idx_ref[0] = jnp.concatenate(idx_cols, axis=1)


def _merge_kernel(vals_ref, idx_ref, seq_ref, seqlp_ref, blps_ref, state_ref,
                  t_ref, seq_out, seqlp_out, sums_out, state_out):
    top_vals = vals_ref[0]
    top_idx = idx_ref[0]
    beam = top_vals.shape[0]
    L = seq_ref.shape[2]

    row_bb = jax.lax.broadcasted_iota(jnp.int32, (beam, beam), 0)
    lane_bb = jax.lax.broadcasted_iota(jnp.int32, (beam, beam), 1)
    bl = blps_ref[0]
    bl_col = jnp.sum(jnp.where(lane_bb == row_bb,
                               jnp.broadcast_to(bl, (beam, beam)), 0.0),
                     axis=1, keepdims=True)
    cand = top_vals + bl_col
    pos = row_bb * beam + lane_bb

    sums_row = jnp.zeros((1, beam), jnp.float32)
    tok_col = jnp.zeros((beam, 1), jnp.int32)
    slp_col = jnp.zeros((beam, 1), jnp.float32)
    src_col = jnp.zeros((beam, 1), jnp.int32)
    lane_1b = jax.lax.broadcasted_iota(jnp.int32, (1, beam), 1)
    row_b1 = jax.lax.broadcasted_iota(jnp.int32, (beam, 1), 0)
    for i in range(beam):
        m = jnp.max(cand)
        p = jnp.min(jnp.where(cand >= m, pos, beam * beam))
        tok = jnp.sum(jnp.where(pos == p, top_idx, 0))
        slp = jnp.sum(jnp.where(pos == p, top_vals, 0.0))
        cand = jnp.where(pos == p, NEG, cand)
        sums_row = jnp.where(lane_1b == i, m, sums_row)
        tok_col = jnp.where(row_b1 == i, tok, tok_col)
        slp_col = jnp.where(row_b1 == i, slp, slp_col)
        src_col = jnp.where(row_b1 == i, p // beam, src_col)

    seq = seq_ref[0]
    seqlp = seqlp_ref[0]
    new_seq = jnp.zeros((beam, L), jnp.int32)
    new_seqlp = jnp.zeros((beam, L), jnp.float32)
    for r in range(beam):
        sel = src_col == r
        new_seq = jnp.where(sel, seq[r:r + 1, :], new_seq)
        new_seqlp = jnp.where(sel, seqlp[r:r + 1, :], new_seqlp)
    col_L = jax.lax.broadcasted_iota(jnp.int32, (beam, L), 1)
    t = t_ref[0, 0]
    new_seq = jnp.where(col_L == t, tok_col, new_seq)
    new_seqlp = jnp.where(col_L == t, slp_col, new_seqlp)

    seq_out[0] = new_seq
    seqlp_out[0] = new_seqlp
    sums_out[0] = sums_row

    layers = state_ref.shape[0]
    for layer in range(layers):
        s = state_ref[layer, 0]
        ns = jnp.zeros(s.shape, jnp.float32)
        for r in range(beam):
            ns = jnp.where(src_col == r, s[r:r + 1, :], ns)
        state_out[layer, 0] = ns


def kernel(logprobs, beam_seq, beam_seq_logprobs, beam_logprobs_sum, state, t):
    B, beam, V = logprobs.shape
    L = beam_seq.shape[2]
    layers, _, _, H = state.shape
    R = B * beam
    G = 8
    ngroups = R // G
    x3 = logprobs.reshape(ngroups, G, V)

    vals_g, idx_g = pl.pallas_call(
        _scan_kernel,
        grid=(ngroups,),
        in_specs=[pl.BlockSpec((1, G, V), lambda g: (g, 0, 0))],
        out_specs=[
            pl.BlockSpec((1, G, beam), lambda g: (g, 0, 0)),
            pl.BlockSpec((1, G, beam), lambda g: (g, 0, 0)),
        ],
        out_shape=[
            jax.ShapeDtypeStruct((ngroups, G, beam), jnp.float32),
            jax.ShapeDtypeStruct((ngroups, G, beam), jnp.int32),
        ],
    )(x3)

    vals_b = vals_g.reshape(B, beam, beam)
    idx_b = idx_g.reshape(B, beam, beam)
    blps3 = beam_logprobs_sum.reshape(B, 1, beam)
    t_arr = jnp.asarray(t, jnp.int32).reshape(1, 1)

    out = pl.pallas_call(
        _merge_kernel,
        grid=(B,),
        in_specs=[
            pl.BlockSpec((1, beam, beam), lambda b: (b, 0, 0)),
            pl.BlockSpec((1, beam, beam), lambda b: (b, 0, 0)),
            pl.BlockSpec((1, beam, L), lambda b: (b, 0, 0)),
            pl.BlockSpec((1, beam, L), lambda b: (b, 0, 0)),
            pl.BlockSpec((1, 1, beam), lambda b: (b, 0, 0)),
            pl.BlockSpec((layers, 1, beam, H), lambda b: (0, b, 0, 0)),
            pl.BlockSpec((1, 1), lambda b: (0, 0)),
        ],
        out_specs=[
            pl.BlockSpec((1, beam, L), lambda b: (b, 0, 0)),
            pl.BlockSpec((1, beam, L), lambda b: (b, 0, 0)),
            pl.BlockSpec((1, 1, beam), lambda b: (b, 0, 0)),
            pl.BlockSpec((layers, 1, beam, H), lambda b: (0, b, 0, 0)),
        ],
        out_shape=[
            jax.ShapeDtypeStruct((B, beam, L), jnp.int32),
            jax.ShapeDtypeStruct((B, beam, L), jnp.float32),
            jax.ShapeDtypeStruct((B, 1, beam), jnp.float32),
            jax.ShapeDtypeStruct((layers, B, beam, H), jnp.float32),
        ],
    )(vals_b, idx_b, beam_seq, beam_seq_logprobs, blps3, state, t_arr)
    new_seq, new_seqlp, sums3, new_state = out
    return (new_seq, new_seqlp, sums3.reshape(B, beam), new_state)


# speedup vs baseline: 121.2461x; 2.9534x over previous
import jax
import jax.numpy as jnp
from jax.experimental import pallas as pl

UNK = 3
NEG = float("-inf")
SEG = 1024


def _lane_max(seg):
    W = seg.shape[1]
    while W > 128 and W % 2 == 0:
        seg = jnp.maximum(seg[:, :W // 2], seg[:, W // 2:])
        W //= 2
    return jnp.max(seg, axis=1, keepdims=True)


def _scan_kernel(x_ref, vals_ref, idx_ref, *, nb, beam, V):
    nseg = (V + SEG - 1) // SEG
    last_w = V - (nseg - 1) * SEG
    for i in range(nb):
        cols = []
        for s in range(nseg):
            lo = s * SEG
            w = SEG if s < nseg - 1 else last_w
            seg = x_ref[i, :, lo:lo + w]
            if s == 0:
                li = jax.lax.broadcasted_iota(jnp.int32, (beam, w), 1)
                seg = jnp.where(li == UNK, seg - 1000.0, seg)
            cols.append(_lane_max(seg))
        segmax = jnp.concatenate(cols, axis=1)

        iota_s = jax.lax.broadcasted_iota(jnp.int32, (beam, nseg), 1)
        sm = segmax
        scols = []
        for _ in range(beam):
            m = jnp.max(sm, axis=1, keepdims=True)
            am = jnp.min(jnp.where(sm >= m, iota_s, nseg),
                         axis=1, keepdims=True)
            scols.append(am)
            sm = jnp.where(iota_s == am, NEG, sm)

        row_b1 = jax.lax.broadcasted_iota(jnp.int32, (beam, 1), 0)
        iota_w = jax.lax.broadcasted_iota(jnp.int32, (1, SEG), 1)
        s_rk = [[jnp.sum(jnp.where(row_b1 == r, scols[k], 0))
                 for k in range(beam)] for r in range(beam)]
        beam_rows = []
        beam_gidx = []
        for r in range(beam):
            y_st = x_ref[i, pl.ds(r, 1), V - SEG:V]
            ys = []
            gls = []
            for k in range(beam):
                s = s_rk[r][k]
                is_last = s >= nseg - 1
                s_dyn = jnp.minimum(s, nseg - 2)
                y_dyn = x_ref[i, pl.ds(r, 1), pl.ds(s_dyn * SEG, SEG)]
                y = jnp.where(is_last, y_st, y_dyn)
                start = jnp.where(is_last, V - SEG, s_dyn * SEG)
                gl = start + iota_w
                y = jnp.where(gl >= s * SEG, y, NEG)
                y = jnp.where(gl == UNK, y - 1000.0, y)
                ys.append(y)
                gls.append(gl)
            beam_rows.append(jnp.concatenate(ys, axis=1))
            beam_gidx.append(jnp.concatenate(gls, axis=1))
        Y = jnp.concatenate(beam_rows, axis=0)
        G = jnp.concatenate(beam_gidx, axis=0)

        v_cols = []
        i_cols = []
        for _ in range(beam):
            m = jnp.max(Y, axis=1, keepdims=True)
            p = jnp.min(jnp.where(Y >= m, G, V), axis=1, keepdims=True)
            v_cols.append(m)
            i_cols.append(p)
            Y = jnp.where(G == p, NEG, Y)
        vals_ref[i] = jnp.concatenate(v_cols, axis=1)
        idx_ref[i] = jnp.concatenate(i_cols, axis=1)


def _merge_kernel(vals_ref, idx_ref, blps_ref, seq_ref, seqlp_ref, state_ref,
                  t_ref, seq_out, seqlp_out, sums_out, state_out):
    B, bb = vals_ref.shape[0], vals_ref.shape[2]
    beam = seq_ref.shape[1]
    L = seq_ref.shape[2]
    layers = state_ref.shape[0]

    vals = vals_ref[:, 0, :]
    idx = idx_ref[:, 0, :]
    pos = jax.lax.broadcasted_iota(jnp.int32, (B, bb), 1)
    cand = vals
    for r in range(beam):
        sel = (pos >= r * beam) & (pos < (r + 1) * beam)
        cand = jnp.where(sel, cand + blps_ref[:, r:r + 1], cand)

    lane_b = jax.lax.broadcasted_iota(jnp.int32, (B, beam), 1)
    col_L = jax.lax.broadcasted_iota(jnp.int32, (B, L), 1)
    t = t_ref[0, 0]
    sums = jnp.zeros((B, beam), jnp.float32)
    for i in range(beam):
        m = jnp.max(cand, axis=1, keepdims=True)
        p = jnp.min(jnp.where(cand >= m, pos, bb), axis=1, keepdims=True)
        tok_i = jnp.sum(jnp.where(pos == p, idx, 0), axis=1, keepdims=True)
        slp_i = jnp.sum(jnp.where(pos == p, vals, 0.0), axis=1, keepdims=True)
        cand = jnp.where(pos == p, NEG, cand)
        src_i = p // beam
        sums = jnp.where(lane_b == i, m, sums)

        ns_i = jnp.zeros((B, L), jnp.int32)
        nslp_i = jnp.zeros((B, L), jnp.float32)
        for r in range(beam):
            sel = src_i == r
            ns_i = jnp.where(sel, seq_ref[:, r, :], ns_i)
            nslp_i = jnp.where(sel, seqlp_ref[:, r, :], nslp_i)
        ns_i = jnp.where(col_L == t, tok_i, ns_i)
        nslp_i = jnp.where(col_L == t, slp_i, nslp_i)
        seq_out[:, i, :] = ns_i
        seqlp_out[:, i, :] = nslp_i
        for layer in range(layers):
            st_i = jnp.zeros(state_ref.shape[1:2] + state_ref.shape[3:],
                             jnp.float32)
            for r in range(beam):
                st_i = jnp.where(src_i == r, state_ref[layer, :, r, :], st_i)
            state_out[layer, :, i, :] = st_i
    sums_out[...] = sums


def kernel(logprobs, beam_seq, beam_seq_logprobs, beam_logprobs_sum, state, t):
    B, beam, V = logprobs.shape
    L = beam_seq.shape[2]
    layers, _, _, H = state.shape
    NB = 4
    t_arr = jnp.asarray(t, jnp.int32).reshape(1, 1)

    import functools
    scan_body = functools.partial(_scan_kernel, nb=NB, beam=beam, V=V)
    vals_g, idx_g = pl.pallas_call(
        scan_body,
        grid=(B // NB,),
        in_specs=[pl.BlockSpec((NB, beam, V), lambda g: (g, 0, 0))],
        out_specs=[
            pl.BlockSpec((NB, beam, beam), lambda g: (g, 0, 0)),
            pl.BlockSpec((NB, beam, beam), lambda g: (g, 0, 0)),
        ],
        out_shape=[
            jax.ShapeDtypeStruct((B, beam, beam), jnp.float32),
            jax.ShapeDtypeStruct((B, beam, beam), jnp.int32),
        ],
    )(logprobs)
    vals_g = vals_g.reshape(B, 1, beam * beam)
    idx_g = idx_g.reshape(B, 1, beam * beam)

    out = pl.pallas_call(
        _merge_kernel,
        grid=(1,),
        in_specs=[
            pl.BlockSpec((B, 1, beam * beam), lambda g: (0, 0, 0)),
            pl.BlockSpec((B, 1, beam * beam), lambda g: (0, 0, 0)),
            pl.BlockSpec((B, beam), lambda g: (0, 0)),
            pl.BlockSpec((B, beam, L), lambda g: (0, 0, 0)),
            pl.BlockSpec((B, beam, L), lambda g: (0, 0, 0)),
            pl.BlockSpec((layers, B, beam, H), lambda g: (0, 0, 0, 0)),
            pl.BlockSpec((1, 1), lambda g: (0, 0)),
        ],
        out_specs=[
            pl.BlockSpec((B, beam, L), lambda g: (0, 0, 0)),
            pl.BlockSpec((B, beam, L), lambda g: (0, 0, 0)),
            pl.BlockSpec((B, beam), lambda g: (0, 0)),
            pl.BlockSpec((layers, B, beam, H), lambda g: (0, 0, 0, 0)),
        ],
        out_shape=[
            jax.ShapeDtypeStruct((B, beam, L), jnp.int32),
            jax.ShapeDtypeStruct((B, beam, L), jnp.float32),
            jax.ShapeDtypeStruct((B, beam), jnp.float32),
            jax.ShapeDtypeStruct((layers, B, beam, H), jnp.float32),
        ],
    )(vals_g, idx_g, beam_logprobs_sum, beam_seq, beam_seq_logprobs, state,
      t_arr)
    new_seq, new_seqlp, sums, new_state = out
    return (new_seq, new_seqlp, sums, new_state)


# speedup vs baseline: 173.4886x; 1.4309x over previous
import jax
import jax.numpy as jnp
from jax.experimental import pallas as pl

UNK = 3
NEG = float("-inf")
SEG = 1024


def _lane_max(seg):
    W = seg.shape[1]
    while W > 128 and W % 2 == 0:
        seg = jnp.maximum(seg[:, :W // 2], seg[:, W // 2:])
        W //= 2
    return jnp.max(seg, axis=1, keepdims=True)


def _scan_kernel(x_ref, vals_ref, idx_ref, *, rows, beam, V):
    nseg = (V + SEG - 1) // SEG
    last_w = V - (nseg - 1) * SEG
    cols = []
    for s in range(nseg):
        lo = s * SEG
        w = SEG if s < nseg - 1 else last_w
        seg = x_ref[0, :, lo:lo + w]
        if s == 0:
            li = jax.lax.broadcasted_iota(jnp.int32, (rows, w), 1)
            seg = jnp.where(li == UNK, seg - 1000.0, seg)
        cols.append(_lane_max(seg))
    segmax = jnp.concatenate(cols, axis=1)

    iota_s = jax.lax.broadcasted_iota(jnp.int32, (rows, nseg), 1)
    sm = segmax
    scols = []
    for _ in range(beam):
        m = jnp.max(sm, axis=1, keepdims=True)
        am = jnp.min(jnp.where(sm >= m, iota_s, nseg),
                     axis=1, keepdims=True)
        scols.append(am)
        sm = jnp.where(iota_s == am, NEG, sm)

    row_b1 = jax.lax.broadcasted_iota(jnp.int32, (rows, 1), 0)
    iota_w = jax.lax.broadcasted_iota(jnp.int32, (1, SEG), 1)
    s_jk = [[jnp.sum(jnp.where(row_b1 == j, scols[k], 0))
             for k in range(beam)] for j in range(rows)]
    y_rows = []
    g_rows = []
    for j in range(rows):
        y_st = x_ref[0, pl.ds(j, 1), V - SEG:V]
        ys = []
        gls = []
        for k in range(beam):
            s = s_jk[j][k]
            is_last = s >= nseg - 1
            s_dyn = jnp.minimum(s, nseg - 2)
            y_dyn = x_ref[0, pl.ds(j, 1), pl.ds(s_dyn * SEG, SEG)]
            y = jnp.where(is_last, y_st, y_dyn)
            start = jnp.where(is_last, V - SEG, s_dyn * SEG)
            gl = start + iota_w
            y = jnp.where(gl >= s * SEG, y, NEG)
            y = jnp.where(gl == UNK, y - 1000.0, y)
            ys.append(y)
            gls.append(gl)
        y_rows.append(jnp.concatenate(ys, axis=1))
        g_rows.append(jnp.concatenate(gls, axis=1))
    Y = jnp.concatenate(y_rows, axis=0)
    G = jnp.concatenate(g_rows, axis=0)

    v_cols = []
    i_cols = []
    for _ in range(beam):
        m = jnp.max(Y, axis=1, keepdims=True)
        p = jnp.min(jnp.where(Y >= m, G, V), axis=1, keepdims=True)
        v_cols.append(m)
        i_cols.append(p)
        Y = jnp.where(G == p, NEG, Y)
    vals_ref[0] = jnp.concatenate(v_cols, axis=1)
    idx_ref[0] = jnp.concatenate(i_cols, axis=1)


def _merge_kernel(vals_ref, idx_ref, blps_ref, seq_ref, seqlp_ref, state_ref,
                  t_ref, seq_out, seqlp_out, sums_out, state_out):
    B, bb = vals_ref.shape[0], vals_ref.shape[2]
    beam = seq_ref.shape[1]
    L = seq_ref.shape[2]
    layers = state_ref.shape[0]

    vals = vals_ref[:, 0, :]
    idx = idx_ref[:, 0, :]
    pos = jax.lax.broadcasted_iota(jnp.int32, (B, bb), 1)
    cand = vals
    for r in range(beam):
        sel = (pos >= r * beam) & (pos < (r + 1) * beam)
        cand = jnp.where(sel, cand + blps_ref[:, r:r + 1], cand)

    lane_b = jax.lax.broadcasted_iota(jnp.int32, (B, beam), 1)
    col_L = jax.lax.broadcasted_iota(jnp.int32, (B, L), 1)
    t = t_ref[0, 0]
    sums = jnp.zeros((B, beam), jnp.float32)
    for i in range(beam):
        m = jnp.max(cand, axis=1, keepdims=True)
        p = jnp.min(jnp.where(cand >= m, pos, bb), axis=1, keepdims=True)
        tok_i = jnp.sum(jnp.where(pos == p, idx, 0), axis=1, keepdims=True)
        slp_i = jnp.sum(jnp.where(pos == p, vals, 0.0), axis=1, keepdims=True)
        cand = jnp.where(pos == p, NEG, cand)
        src_i = p // beam
        sums = jnp.where(lane_b == i, m, sums)

        ns_i = jnp.zeros((B, L), jnp.int32)
        nslp_i = jnp.zeros((B, L), jnp.float32)
        for r in range(beam):
            sel = src_i == r
            ns_i = jnp.where(sel, seq_ref[:, r, :], ns_i)
            nslp_i = jnp.where(sel, seqlp_ref[:, r, :], nslp_i)
        ns_i = jnp.where(col_L == t, tok_i, ns_i)
        nslp_i = jnp.where(col_L == t, slp_i, nslp_i)
        seq_out[:, i, :] = ns_i
        seqlp_out[:, i, :] = nslp_i
        for layer in range(layers):
            st_i = jnp.zeros(state_ref.shape[1:2] + state_ref.shape[3:],
                             jnp.float32)
            for r in range(beam):
                st_i = jnp.where(src_i == r, state_ref[layer, :, r, :], st_i)
            state_out[layer, :, i, :] = st_i
    sums_out[...] = sums


def kernel(logprobs, beam_seq, beam_seq_logprobs, beam_logprobs_sum, state, t):
    B, beam, V = logprobs.shape
    L = beam_seq.shape[2]
    layers, _, _, H = state.shape
    RB = 8
    t_arr = jnp.asarray(t, jnp.int32).reshape(1, 1)
    xt = jnp.transpose(logprobs, (1, 0, 2))

    import functools
    scan_body = functools.partial(_scan_kernel, rows=RB, beam=beam, V=V)
    vals_t, idx_t = pl.pallas_call(
        scan_body,
        grid=(beam, B // RB),
        in_specs=[pl.BlockSpec((1, RB, V), lambda r, g: (r, g, 0))],
        out_specs=[
            pl.BlockSpec((1, RB, beam), lambda r, g: (r, g, 0)),
            pl.BlockSpec((1, RB, beam), lambda r, g: (r, g, 0)),
        ],
        out_shape=[
            jax.ShapeDtypeStruct((beam, B, beam), jnp.float32),
            jax.ShapeDtypeStruct((beam, B, beam), jnp.int32),
        ],
    )(xt)
    vals_g = vals_t.transpose(1, 0, 2).reshape(B, 1, beam * beam)
    idx_g = idx_t.transpose(1, 0, 2).reshape(B, 1, beam * beam)

    out = pl.pallas_call(
        _merge_kernel,
        grid=(1,),
        in_specs=[
            pl.BlockSpec((B, 1, beam * beam), lambda g: (0, 0, 0)),
            pl.BlockSpec((B, 1, beam * beam), lambda g: (0, 0, 0)),
            pl.BlockSpec((B, beam), lambda g: (0, 0)),
            pl.BlockSpec((B, beam, L), lambda g: (0, 0, 0)),
            pl.BlockSpec((B, beam, L), lambda g: (0, 0, 0)),
            pl.BlockSpec((layers, B, beam, H), lambda g: (0, 0, 0, 0)),
            pl.BlockSpec((1, 1), lambda g: (0, 0)),
        ],
        out_specs=[
            pl.BlockSpec((B, beam, L), lambda g: (0, 0, 0)),
            pl.BlockSpec((B, beam, L), lambda g: (0, 0, 0)),
            pl.BlockSpec((B, beam), lambda g: (0, 0)),
            pl.BlockSpec((layers, B, beam, H), lambda g: (0, 0, 0, 0)),
        ],
        out_shape=[
            jax.ShapeDtypeStruct((B, beam, L), jnp.int32),
            jax.ShapeDtypeStruct((B, beam, L), jnp.float32),
            jax.ShapeDtypeStruct((B, beam), jnp.float32),
            jax.ShapeDtypeStruct((layers, B, beam, H), jnp.float32),
        ],
    )(vals_g, idx_g, beam_logprobs_sum, beam_seq, beam_seq_logprobs, state,
      t_arr)
    new_seq, new_seqlp, sums, new_state = out
    return (new_seq, new_seqlp, sums, new_state)


# speedup vs baseline: 291.5485x; 1.6805x over previous
import jax
import jax.numpy as jnp
from jax.experimental import pallas as pl

UNK = 3
NEG = float("-inf")
SEG = 1024


def _lane_max(seg):
    W = seg.shape[1]
    while W > 128 and W % 2 == 0:
        seg = jnp.maximum(seg[:, :W // 2], seg[:, W // 2:])
        W //= 2
    return jnp.max(seg, axis=1, keepdims=True)


def _scan_kernel(x_ref, vals_ref, idx_ref, *, rows, beam, V):
    nseg = (V + SEG - 1) // SEG
    last_w = V - (nseg - 1) * SEG
    cols = []
    for s in range(nseg):
        lo = s * SEG
        w = SEG if s < nseg - 1 else last_w
        seg = x_ref[0, :, lo:lo + w]
        if s == 0:
            li = jax.lax.broadcasted_iota(jnp.int32, (rows, w), 1)
            seg = jnp.where(li == UNK, seg - 1000.0, seg)
        cols.append(_lane_max(seg))
    segmax = jnp.concatenate(cols, axis=1)

    iota_s = jax.lax.broadcasted_iota(jnp.int32, (rows, nseg), 1)
    sm = segmax
    scols = []
    for _ in range(beam):
        m = jnp.max(sm, axis=1, keepdims=True)
        am = jnp.min(jnp.where(sm >= m, iota_s, nseg),
                     axis=1, keepdims=True)
        scols.append(am)
        sm = jnp.where(iota_s == am, NEG, sm)

    row_b1 = jax.lax.broadcasted_iota(jnp.int32, (rows, 1), 0)
    iota_w = jax.lax.broadcasted_iota(jnp.int32, (1, SEG), 1)
    s_jk = [[jnp.sum(jnp.where(row_b1 == j, scols[k], 0))
             for k in range(beam)] for j in range(rows)]
    y_rows = []
    g_rows = []
    for j in range(rows):
        y_st = x_ref[0, pl.ds(j, 1), V - SEG:V]
        ys = []
        gls = []
        for k in range(beam):
            s = s_jk[j][k]
            is_last = s >= nseg - 1
            s_dyn = jnp.minimum(s, nseg - 2)
            y_dyn = x_ref[0, pl.ds(j, 1), pl.ds(s_dyn * SEG, SEG)]
            y = jnp.where(is_last, y_st, y_dyn)
            start = jnp.where(is_last, V - SEG, s_dyn * SEG)
            gl = start + iota_w
            y = jnp.where(gl >= s * SEG, y, NEG)
            y = jnp.where(gl == UNK, y - 1000.0, y)
            ys.append(y)
            gls.append(gl)
        y_rows.append(jnp.concatenate(ys, axis=1))
        g_rows.append(jnp.concatenate(gls, axis=1))
    Y = jnp.concatenate(y_rows, axis=0)
    G = jnp.concatenate(g_rows, axis=0)

    v_cols = []
    i_cols = []
    for _ in range(beam):
        m = jnp.max(Y, axis=1, keepdims=True)
        p = jnp.min(jnp.where(Y >= m, G, V), axis=1, keepdims=True)
        v_cols.append(m)
        i_cols.append(p)
        Y = jnp.where(G == p, NEG, Y)
    vals_ref[0] = jnp.concatenate(v_cols, axis=1)
    idx_ref[0] = jnp.concatenate(i_cols, axis=1)


def _merge_kernel(vals_ref, idx_ref, blps_ref, seq_ref, seqlp_ref, state_ref,
                  t_ref, seq_out, seqlp_out, sums_out, state_out):
    B, bb = vals_ref.shape[0], vals_ref.shape[2]
    beam = seq_ref.shape[1]
    L = seq_ref.shape[2]
    layers = state_ref.shape[0]

    vals = vals_ref[:, 0, :]
    idx = idx_ref[:, 0, :]
    pos = jax.lax.broadcasted_iota(jnp.int32, (B, bb), 1)
    cand = vals
    for r in range(beam):
        sel = (pos >= r * beam) & (pos < (r + 1) * beam)
        cand = jnp.where(sel, cand + blps_ref[:, r:r + 1], cand)

    lane_b = jax.lax.broadcasted_iota(jnp.int32, (B, beam), 1)
    col_L = jax.lax.broadcasted_iota(jnp.int32, (B, L), 1)
    t = t_ref[0, 0]
    sums = jnp.zeros((B, beam), jnp.float32)
    for i in range(beam):
        m = jnp.max(cand, axis=1, keepdims=True)
        p = jnp.min(jnp.where(cand >= m, pos, bb), axis=1, keepdims=True)
        tok_i = jnp.sum(jnp.where(pos == p, idx, 0), axis=1, keepdims=True)
        slp_i = jnp.sum(jnp.where(pos == p, vals, 0.0), axis=1, keepdims=True)
        cand = jnp.where(pos == p, NEG, cand)
        src_i = p // beam
        sums = jnp.where(lane_b == i, m, sums)

        ns_i = jnp.zeros((B, L), jnp.int32)
        nslp_i = jnp.zeros((B, L), jnp.float32)
        for r in range(beam):
            sel = src_i == r
            ns_i = jnp.where(sel, seq_ref[:, r, :], ns_i)
            nslp_i = jnp.where(sel, seqlp_ref[:, r, :], nslp_i)
        ns_i = jnp.where(col_L == t, tok_i, ns_i)
        nslp_i = jnp.where(col_L == t, slp_i, nslp_i)
        seq_out[:, i, :] = ns_i
        seqlp_out[:, i, :] = nslp_i
        for layer in range(layers):
            st_i = jnp.zeros(state_ref.shape[1:2] + state_ref.shape[3:],
                             jnp.float32)
            for r in range(beam):
                st_i = jnp.where(src_i == r, state_ref[layer, :, r, :], st_i)
            state_out[layer, :, i, :] = st_i
    sums_out[...] = sums


def kernel(logprobs, beam_seq, beam_seq_logprobs, beam_logprobs_sum, state, t):
    B, beam, V = logprobs.shape
    L = beam_seq.shape[2]
    layers, _, _, H = state.shape
    RB = 32
    t_arr = jnp.asarray(t, jnp.int32).reshape(1, 1)
    xt = jnp.transpose(logprobs, (1, 0, 2))

    import functools
    scan_body = functools.partial(_scan_kernel, rows=RB, beam=beam, V=V)
    vals_t, idx_t = pl.pallas_call(
        scan_body,
        grid=(beam, B // RB),
        in_specs=[pl.BlockSpec((1, RB, V), lambda r, g: (r, g, 0))],
        out_specs=[
            pl.BlockSpec((1, RB, beam), lambda r, g: (r, g, 0)),
            pl.BlockSpec((1, RB, beam), lambda r, g: (r, g, 0)),
        ],
        out_shape=[
            jax.ShapeDtypeStruct((beam, B, beam), jnp.float32),
            jax.ShapeDtypeStruct((beam, B, beam), jnp.int32),
        ],
    )(xt)
    vals_g = vals_t.transpose(1, 0, 2).reshape(B, 1, beam * beam)
    idx_g = idx_t.transpose(1, 0, 2).reshape(B, 1, beam * beam)

    out = pl.pallas_call(
        _merge_kernel,
        grid=(1,),
        in_specs=[
            pl.BlockSpec((B, 1, beam * beam), lambda g: (0, 0, 0)),
            pl.BlockSpec((B, 1, beam * beam), lambda g: (0, 0, 0)),
            pl.BlockSpec((B, beam), lambda g: (0, 0)),
            pl.BlockSpec((B, beam, L), lambda g: (0, 0, 0)),
            pl.BlockSpec((B, beam, L), lambda g: (0, 0, 0)),
            pl.BlockSpec((layers, B, beam, H), lambda g: (0, 0, 0, 0)),
            pl.BlockSpec((1, 1), lambda g: (0, 0)),
        ],
        out_specs=[
            pl.BlockSpec((B, beam, L), lambda g: (0, 0, 0)),
            pl.BlockSpec((B, beam, L), lambda g: (0, 0, 0)),
            pl.BlockSpec((B, beam), lambda g: (0, 0)),
            pl.BlockSpec((layers, B, beam, H), lambda g: (0, 0, 0, 0)),
        ],
        out_shape=[
            jax.ShapeDtypeStruct((B, beam, L), jnp.int32),
            jax.ShapeDtypeStruct((B, beam, L), jnp.float32),
            jax.ShapeDtypeStruct((B, beam), jnp.float32),
            jax.ShapeDtypeStruct((layers, B, beam, H), jnp.float32),
        ],
    )(vals_g, idx_g, beam_logprobs_sum, beam_seq, beam_seq_logprobs, state,
      t_arr)
    new_seq, new_seqlp, sums, new_state = out
    return (new_seq, new_seqlp, sums, new_state)


# speedup vs baseline: 327.9244x; 1.1248x over previous
import jax
import jax.numpy as jnp
from jax.experimental import pallas as pl

UNK = 3
NEG = float("-inf")
SEG = 512


def _lane_max(seg):
    W = seg.shape[1]
    while W > 128 and W % 2 == 0:
        seg = jnp.maximum(seg[:, :W // 2], seg[:, W // 2:])
        W //= 2
    return jnp.max(seg, axis=1, keepdims=True)


def _scan_kernel(x_ref, vals_ref, idx_ref, *, rows, beam, V):
    nseg = (V + SEG - 1) // SEG
    last_w = V - (nseg - 1) * SEG
    cols = []
    for s in range(nseg):
        lo = s * SEG
        w = SEG if s < nseg - 1 else last_w
        seg = x_ref[0, :, lo:lo + w]
        if s == 0:
            li = jax.lax.broadcasted_iota(jnp.int32, (rows, w), 1)
            seg = jnp.where(li == UNK, seg - 1000.0, seg)
        cols.append(_lane_max(seg))
    segmax = jnp.concatenate(cols, axis=1)

    iota_s = jax.lax.broadcasted_iota(jnp.int32, (rows, nseg), 1)
    sm = segmax
    scols = []
    for _ in range(beam):
        m = jnp.max(sm, axis=1, keepdims=True)
        am = jnp.min(jnp.where(sm >= m, iota_s, nseg),
                     axis=1, keepdims=True)
        scols.append(am)
        sm = jnp.where(iota_s == am, NEG, sm)

    row_b1 = jax.lax.broadcasted_iota(jnp.int32, (rows, 1), 0)
    iota_w = jax.lax.broadcasted_iota(jnp.int32, (1, SEG), 1)
    s_jk = [[jnp.sum(jnp.where(row_b1 == j, scols[k], 0))
             for k in range(beam)] for j in range(rows)]
    y_rows = []
    g_rows = []
    for j in range(rows):
        y_st = x_ref[0, pl.ds(j, 1), V - SEG:V]
        ys = []
        gls = []
        for k in range(beam):
            s = s_jk[j][k]
            is_last = s >= nseg - 1
            s_dyn = jnp.minimum(s, nseg - 2)
            y_dyn = x_ref[0, pl.ds(j, 1), pl.ds(s_dyn * SEG, SEG)]
            y = jnp.where(is_last, y_st, y_dyn)
            start = jnp.where(is_last, V - SEG, s_dyn * SEG)
            gl = start + iota_w
            y = jnp.where(gl >= s * SEG, y, NEG)
            y = jnp.where(gl == UNK, y - 1000.0, y)
            ys.append(y)
            gls.append(gl)
        y_rows.append(jnp.concatenate(ys, axis=1))
        g_rows.append(jnp.concatenate(gls, axis=1))
    Y = jnp.concatenate(y_rows, axis=0)
    G = jnp.concatenate(g_rows, axis=0)

    v_cols = []
    i_cols = []
    for _ in range(beam):
        m = jnp.max(Y, axis=1, keepdims=True)
        p = jnp.min(jnp.where(Y >= m, G, V), axis=1, keepdims=True)
        v_cols.append(m)
        i_cols.append(p)
        Y = jnp.where(G == p, NEG, Y)
    vals_ref[0] = jnp.concatenate(v_cols, axis=1)
    idx_ref[0] = jnp.concatenate(i_cols, axis=1)


def _merge_kernel(vals_ref, idx_ref, blps_ref, seq_ref, seqlp_ref, state_ref,
                  t_ref, seq_out, seqlp_out, sums_out, state_out):
    B, bb = vals_ref.shape[0], vals_ref.shape[2]
    beam = seq_ref.shape[1]
    L = seq_ref.shape[2]
    layers = state_ref.shape[0]

    vals = vals_ref[:, 0, :]
    idx = idx_ref[:, 0, :]
    pos = jax.lax.broadcasted_iota(jnp.int32, (B, bb), 1)
    cand = vals
    for r in range(beam):
        sel = (pos >= r * beam) & (pos < (r + 1) * beam)
        cand = jnp.where(sel, cand + blps_ref[:, r:r + 1], cand)

    lane_b = jax.lax.broadcasted_iota(jnp.int32, (B, beam), 1)
    col_L = jax.lax.broadcasted_iota(jnp.int32, (B, L), 1)
    t = t_ref[0, 0]
    sums = jnp.zeros((B, beam), jnp.float32)
    for i in range(beam):
        m = jnp.max(cand, axis=1, keepdims=True)
        p = jnp.min(jnp.where(cand >= m, pos, bb), axis=1, keepdims=True)
        tok_i = jnp.sum(jnp.where(pos == p, idx, 0), axis=1, keepdims=True)
        slp_i = jnp.sum(jnp.where(pos == p, vals, 0.0), axis=1, keepdims=True)
        cand = jnp.where(pos == p, NEG, cand)
        src_i = p // beam
        sums = jnp.where(lane_b == i, m, sums)

        ns_i = jnp.zeros((B, L), jnp.int32)
        nslp_i = jnp.zeros((B, L), jnp.float32)
        for r in range(beam):
            sel = src_i == r
            ns_i = jnp.where(sel, seq_ref[:, r, :], ns_i)
            nslp_i = jnp.where(sel, seqlp_ref[:, r, :], nslp_i)
        ns_i = jnp.where(col_L == t, tok_i, ns_i)
        nslp_i = jnp.where(col_L == t, slp_i, nslp_i)
        seq_out[:, i, :] = ns_i
        seqlp_out[:, i, :] = nslp_i
        for layer in range(layers):
            st_i = jnp.zeros(state_ref.shape[1:2] + state_ref.shape[3:],
                             jnp.float32)
            for r in range(beam):
                st_i = jnp.where(src_i == r, state_ref[layer, :, r, :], st_i)
            state_out[layer, :, i, :] = st_i
    sums_out[...] = sums


def kernel(logprobs, beam_seq, beam_seq_logprobs, beam_logprobs_sum, state, t):
    B, beam, V = logprobs.shape
    L = beam_seq.shape[2]
    layers, _, _, H = state.shape
    RB = 32
    t_arr = jnp.asarray(t, jnp.int32).reshape(1, 1)
    xt = jnp.transpose(logprobs, (1, 0, 2))

    import functools
    scan_body = functools.partial(_scan_kernel, rows=RB, beam=beam, V=V)
    vals_t, idx_t = pl.pallas_call(
        scan_body,
        grid=(beam, B // RB),
        in_specs=[pl.BlockSpec((1, RB, V), lambda r, g: (r, g, 0))],
        out_specs=[
            pl.BlockSpec((1, RB, beam), lambda r, g: (r, g, 0)),
            pl.BlockSpec((1, RB, beam), lambda r, g: (r, g, 0)),
        ],
        out_shape=[
            jax.ShapeDtypeStruct((beam, B, beam), jnp.float32),
            jax.ShapeDtypeStruct((beam, B, beam), jnp.int32),
        ],
    )(xt)
    vals_g = vals_t.transpose(1, 0, 2).reshape(B, 1, beam * beam)
    idx_g = idx_t.transpose(1, 0, 2).reshape(B, 1, beam * beam)

    out = pl.pallas_call(
        _merge_kernel,
        grid=(1,),
        in_specs=[
            pl.BlockSpec((B, 1, beam * beam), lambda g: (0, 0, 0)),
            pl.BlockSpec((B, 1, beam * beam), lambda g: (0, 0, 0)),
            pl.BlockSpec((B, beam), lambda g: (0, 0)),
            pl.BlockSpec((B, beam, L), lambda g: (0, 0, 0)),
            pl.BlockSpec((B, beam, L), lambda g: (0, 0, 0)),
            pl.BlockSpec((layers, B, beam, H), lambda g: (0, 0, 0, 0)),
            pl.BlockSpec((1, 1), lambda g: (0, 0)),
        ],
        out_specs=[
            pl.BlockSpec((B, beam, L), lambda g: (0, 0, 0)),
            pl.BlockSpec((B, beam, L), lambda g: (0, 0, 0)),
            pl.BlockSpec((B, beam), lambda g: (0, 0)),
            pl.BlockSpec((layers, B, beam, H), lambda g: (0, 0, 0, 0)),
        ],
        out_shape=[
            jax.ShapeDtypeStruct((B, beam, L), jnp.int32),
            jax.ShapeDtypeStruct((B, beam, L), jnp.float32),
            jax.ShapeDtypeStruct((B, beam), jnp.float32),
            jax.ShapeDtypeStruct((layers, B, beam, H), jnp.float32),
        ],
    )(vals_g, idx_g, beam_logprobs_sum, beam_seq, beam_seq_logprobs, state,
      t_arr)
    new_seq, new_seqlp, sums, new_state = out
    return (new_seq, new_seqlp, sums, new_state)


# speedup vs baseline: 359.1616x; 1.0953x over previous
import jax
import jax.numpy as jnp
from jax.experimental import pallas as pl

UNK = 3
NEG = float("-inf")
SEG = 512


def _lane_max(seg):
    W = seg.shape[1]
    while W > 128 and W % 2 == 0:
        seg = jnp.maximum(seg[:, :W // 2], seg[:, W // 2:])
        W //= 2
    return jnp.max(seg, axis=1, keepdims=True)


def _scan_kernel(x_ref, vals_ref, idx_ref, *, rows, beam, V):
    nseg = (V + SEG - 1) // SEG
    last_w = V - (nseg - 1) * SEG
    cols = []
    for s in range(nseg):
        lo = s * SEG
        w = SEG if s < nseg - 1 else last_w
        seg = x_ref[0, :, lo:lo + w]
        if s == 0:
            li = jax.lax.broadcasted_iota(jnp.int32, (rows, w), 1)
            seg = jnp.where(li == UNK, seg - 1000.0, seg)
        cols.append(_lane_max(seg))
    segmax = jnp.concatenate(cols, axis=1)

    iota_s = jax.lax.broadcasted_iota(jnp.int32, (rows, nseg), 1)
    sm = segmax
    scols = []
    for _ in range(beam):
        m = jnp.max(sm, axis=1, keepdims=True)
        am = jnp.min(jnp.where(sm >= m, iota_s, nseg),
                     axis=1, keepdims=True)
        scols.append(am)
        sm = jnp.where(iota_s == am, NEG, sm)

    row_b1 = jax.lax.broadcasted_iota(jnp.int32, (rows, 1), 0)
    iota_w = jax.lax.broadcasted_iota(jnp.int32, (1, SEG), 1)
    s_jk = [[jnp.sum(jnp.where(row_b1 == j, scols[k], 0))
             for k in range(beam)] for j in range(rows)]
    y_rows = []
    g_rows = []
    for j in range(rows):
        y_st = x_ref[0, pl.ds(j, 1), V - SEG:V]
        ys = []
        gls = []
        for k in range(beam):
            s = s_jk[j][k]
            is_last = s >= nseg - 1
            s_dyn = jnp.minimum(s, nseg - 2)
            y_dyn = x_ref[0, pl.ds(j, 1), pl.ds(s_dyn * SEG, SEG)]
            y = jnp.where(is_last, y_st, y_dyn)
            start = jnp.where(is_last, V - SEG, s_dyn * SEG)
            gl = start + iota_w
            y = jnp.where(gl >= s * SEG, y, NEG)
            y = jnp.where(gl == UNK, y - 1000.0, y)
            ys.append(y)
            gls.append(gl)
        y_rows.append(jnp.concatenate(ys, axis=1))
        g_rows.append(jnp.concatenate(gls, axis=1))
    Y = jnp.concatenate(y_rows, axis=0)
    G = jnp.concatenate(g_rows, axis=0)

    v_cols = []
    i_cols = []
    for _ in range(beam):
        m = jnp.max(Y, axis=1, keepdims=True)
        p = jnp.min(jnp.where(Y >= m, G, V), axis=1, keepdims=True)
        v_cols.append(m)
        i_cols.append(p)
        Y = jnp.where(G == p, NEG, Y)
    vals_ref[0] = jnp.concatenate(v_cols, axis=1)
    idx_ref[0] = jnp.concatenate(i_cols, axis=1)


def _merge_kernel(vals_ref, idx_ref, blps_ref, seq_ref, seqlp_ref, state_ref,
                  t_ref, seq_out, seqlp_out, sums_out, state_out):
    B, bb = vals_ref.shape[0], vals_ref.shape[2]
    beam = seq_ref.shape[1]
    L = seq_ref.shape[2]
    layers = state_ref.shape[0]

    vals = vals_ref[:, 0, :]
    idx = idx_ref[:, 0, :]
    pos = jax.lax.broadcasted_iota(jnp.int32, (B, bb), 1)
    cand = vals
    for r in range(beam):
        sel = (pos >= r * beam) & (pos < (r + 1) * beam)
        cand = jnp.where(sel, cand + blps_ref[:, r:r + 1], cand)

    lane_b = jax.lax.broadcasted_iota(jnp.int32, (B, beam), 1)
    col_L = jax.lax.broadcasted_iota(jnp.int32, (B, L), 1)
    t = t_ref[0, 0]
    sums = jnp.zeros((B, beam), jnp.float32)
    for i in range(beam):
        m = jnp.max(cand, axis=1, keepdims=True)
        p = jnp.min(jnp.where(cand >= m, pos, bb), axis=1, keepdims=True)
        tok_i = jnp.sum(jnp.where(pos == p, idx, 0), axis=1, keepdims=True)
        slp_i = jnp.sum(jnp.where(pos == p, vals, 0.0), axis=1, keepdims=True)
        cand = jnp.where(pos == p, NEG, cand)
        src_i = p // beam
        sums = jnp.where(lane_b == i, m, sums)

        ns_i = jnp.zeros((B, L), jnp.int32)
        nslp_i = jnp.zeros((B, L), jnp.float32)
        for r in range(beam):
            sel = src_i == r
            ns_i = jnp.where(sel, seq_ref[:, r, :], ns_i)
            nslp_i = jnp.where(sel, seqlp_ref[:, r, :], nslp_i)
        ns_i = jnp.where(col_L == t, tok_i, ns_i)
        nslp_i = jnp.where(col_L == t, slp_i, nslp_i)
        seq_out[:, i, :] = ns_i
        seqlp_out[:, i, :] = nslp_i
        for layer in range(layers):
            st_i = jnp.zeros(state_ref.shape[2:], jnp.float32)
            for r in range(beam):
                st_i = jnp.where(src_i == r, state_ref[layer, r, :, :], st_i)
            state_out[layer, i, :, :] = st_i
    sums_out[...] = sums


def kernel(logprobs, beam_seq, beam_seq_logprobs, beam_logprobs_sum, state, t):
    B, beam, V = logprobs.shape
    L = beam_seq.shape[2]
    layers, _, _, H = state.shape
    RB = 32
    t_arr = jnp.asarray(t, jnp.int32).reshape(1, 1)
    xt = jnp.transpose(logprobs, (1, 0, 2))

    import functools
    scan_body = functools.partial(_scan_kernel, rows=RB, beam=beam, V=V)
    vals_t, idx_t = pl.pallas_call(
        scan_body,
        grid=(beam, B // RB),
        in_specs=[pl.BlockSpec((1, RB, V), lambda r, g: (r, g, 0))],
        out_specs=[
            pl.BlockSpec((1, RB, beam), lambda r, g: (r, g, 0)),
            pl.BlockSpec((1, RB, beam), lambda r, g: (r, g, 0)),
        ],
        out_shape=[
            jax.ShapeDtypeStruct((beam, B, beam), jnp.float32),
            jax.ShapeDtypeStruct((beam, B, beam), jnp.int32),
        ],
    )(xt)
    vals_g = vals_t.transpose(1, 0, 2).reshape(B, 1, beam * beam)
    idx_g = idx_t.transpose(1, 0, 2).reshape(B, 1, beam * beam)

    out = pl.pallas_call(
        _merge_kernel,
        grid=(1,),
        in_specs=[
            pl.BlockSpec((B, 1, beam * beam), lambda g: (0, 0, 0)),
            pl.BlockSpec((B, 1, beam * beam), lambda g: (0, 0, 0)),
            pl.BlockSpec((B, beam), lambda g: (0, 0)),
            pl.BlockSpec((B, beam, L), lambda g: (0, 0, 0)),
            pl.BlockSpec((B, beam, L), lambda g: (0, 0, 0)),
            pl.BlockSpec((layers, beam, B, H), lambda g: (0, 0, 0, 0)),
            pl.BlockSpec((1, 1), lambda g: (0, 0)),
        ],
        out_specs=[
            pl.BlockSpec((B, beam, L), lambda g: (0, 0, 0)),
            pl.BlockSpec((B, beam, L), lambda g: (0, 0, 0)),
            pl.BlockSpec((B, beam), lambda g: (0, 0)),
            pl.BlockSpec((layers, beam, B, H), lambda g: (0, 0, 0, 0)),
        ],
        out_shape=[
            jax.ShapeDtypeStruct((B, beam, L), jnp.int32),
            jax.ShapeDtypeStruct((B, beam, L), jnp.float32),
            jax.ShapeDtypeStruct((B, beam), jnp.float32),
            jax.ShapeDtypeStruct((layers, beam, B, H), jnp.float32),
        ],
    )(vals_g, idx_g, beam_logprobs_sum, beam_seq, beam_seq_logprobs,
      jnp.transpose(state, (0, 2, 1, 3)), t_arr)
    new_seq, new_seqlp, sums, new_state_t = out
    return (new_seq, new_seqlp, sums, jnp.transpose(new_state_t, (0, 2, 1, 3)))


# speedup vs baseline: 420.3397x; 1.1703x over previous
import jax
import jax.numpy as jnp
from jax.experimental import pallas as pl

UNK = 3
NEG = float("-inf")
SEG = 512


def _lane_max(seg):
    W = seg.shape[1]
    while W > 128 and W % 2 == 0:
        seg = jnp.maximum(seg[:, :W // 2], seg[:, W // 2:])
        W //= 2
    return jnp.max(seg, axis=1, keepdims=True)


def _scan_kernel(x_ref, vals_ref, idx_ref, *, rows, beam, V):
    nseg = (V + SEG - 1) // SEG
    last_w = V - (nseg - 1) * SEG
    cols = []
    for s in range(nseg):
        lo = s * SEG
        w = SEG if s < nseg - 1 else last_w
        seg = x_ref[0, :, lo:lo + w]
        if s == 0:
            li = jax.lax.broadcasted_iota(jnp.int32, (rows, w), 1)
            seg = jnp.where(li == UNK, seg - 1000.0, seg)
        cols.append(_lane_max(seg))
    segmax = jnp.concatenate(cols, axis=1)

    iota_s = jax.lax.broadcasted_iota(jnp.int32, (rows, nseg), 1)
    sm = segmax
    scols = []
    for _ in range(beam):
        m = jnp.max(sm, axis=1, keepdims=True)
        am = jnp.min(jnp.where(sm >= m, iota_s, nseg),
                     axis=1, keepdims=True)
        scols.append(am)
        sm = jnp.where(iota_s == am, NEG, sm)

    row_b1 = jax.lax.broadcasted_iota(jnp.int32, (rows, 1), 0)
    iota_w = jax.lax.broadcasted_iota(jnp.int32, (1, SEG), 1)
    s_jk = [[jnp.sum(jnp.where(row_b1 == j, scols[k], 0))
             for k in range(beam)] for j in range(rows)]
    y_rows = []
    g_rows = []
    for j in range(rows):
        y_st = x_ref[0, pl.ds(j, 1), V - SEG:V]
        ys = []
        gls = []
        for k in range(beam):
            s = s_jk[j][k]
            is_last = s >= nseg - 1
            s_dyn = jnp.minimum(s, nseg - 2)
            y_dyn = x_ref[0, pl.ds(j, 1), pl.ds(s_dyn * SEG, SEG)]
            y = jnp.where(is_last, y_st, y_dyn)
            start = jnp.where(is_last, V - SEG, s_dyn * SEG)
            gl = start + iota_w
            y = jnp.where(gl >= s * SEG, y, NEG)
            y = jnp.where(gl == UNK, y - 1000.0, y)
            ys.append(y)
            gls.append(gl)
        y_rows.append(jnp.concatenate(ys, axis=1))
        g_rows.append(jnp.concatenate(gls, axis=1))
    Y = jnp.concatenate(y_rows, axis=0)
    G = jnp.concatenate(g_rows, axis=0)

    v_cols = []
    i_cols = []
    for _ in range(beam):
        m = jnp.max(Y, axis=1, keepdims=True)
        p = jnp.min(jnp.where(Y >= m, G, V), axis=1, keepdims=True)
        v_cols.append(m)
        i_cols.append(p)
        Y = jnp.where(G == p, NEG, Y)
    vals_ref[0] = jnp.concatenate(v_cols, axis=1)
    idx_ref[0] = jnp.concatenate(i_cols, axis=1)


def _merge_kernel(vals_ref, idx_ref, blps_ref, seq_ref, seqlp_ref, state_ref,
                  t_ref, seq_out, seqlp_out, sums_out, state_out):
    beam, B, _ = vals_ref.shape
    L = seq_ref.shape[1]
    layers = state_ref.shape[0]
    bb = beam * beam

    vals = jnp.concatenate([vals_ref[r] for r in range(beam)], axis=1)
    idx = jnp.concatenate([idx_ref[r] for r in range(beam)], axis=1)
    pos = jax.lax.broadcasted_iota(jnp.int32, (B, bb), 1)
    cand = vals
    for r in range(beam):
        sel = (pos >= r * beam) & (pos < (r + 1) * beam)
        cand = jnp.where(sel, cand + blps_ref[:, r:r + 1], cand)

    eye = jnp.where(
        jax.lax.broadcasted_iota(jnp.int32, (B, B), 0)
        == jax.lax.broadcasted_iota(jnp.int32, (B, B), 1), 1.0, 0.0)

    def rowify(col):
        return jax.lax.dot_general(col, eye, (((0,), (0,)), ((), ())),
                                   preferred_element_type=jnp.float32)

    row_L = jax.lax.broadcasted_iota(jnp.int32, (L, 1), 0)
    row_b = jax.lax.broadcasted_iota(jnp.int32, (beam, 1), 0)
    t = t_ref[0, 0]
    sums_t = jnp.zeros((beam, B), jnp.float32)
    for i in range(beam):
        m = jnp.max(cand, axis=1, keepdims=True)
        p = jnp.min(jnp.where(cand >= m, pos, bb), axis=1, keepdims=True)
        tok_i = jnp.sum(jnp.where(pos == p, idx, 0), axis=1, keepdims=True)
        slp_i = jnp.sum(jnp.where(pos == p, vals, 0.0), axis=1, keepdims=True)
        cand = jnp.where(pos == p, NEG, cand)
        src_i = p // beam
        src_row = rowify(src_i.astype(jnp.float32))
        tok_row = rowify(tok_i.astype(jnp.float32))
        slp_row = rowify(slp_i)
        m_row = rowify(m)
        sums_t = jnp.where(row_b == i, m_row, sums_t)

        ns_i = jnp.zeros((L, B), jnp.float32)
        nslp_i = jnp.zeros((L, B), jnp.float32)
        for r in range(beam):
            sel = (src_row >= r - 0.5) & (src_row <= r + 0.5)
            ns_i = jnp.where(sel, seq_ref[r].astype(jnp.float32), ns_i)
            nslp_i = jnp.where(sel, seqlp_ref[r], nslp_i)
        ns_i = jnp.where(row_L == t, tok_row, ns_i)
        nslp_i = jnp.where(row_L == t, slp_row, nslp_i)
        seq_out[i] = jnp.round(ns_i).astype(jnp.int32)
        seqlp_out[i] = nslp_i
        for layer in range(layers):
            st_i = jnp.zeros(state_ref.shape[2:], jnp.float32)
            for r in range(beam):
                st_i = jnp.where(src_i == r, state_ref[layer, r, :, :], st_i)
            state_out[layer, i, :, :] = st_i
    sums_out[...] = sums_t


def kernel(logprobs, beam_seq, beam_seq_logprobs, beam_logprobs_sum, state, t):
    B, beam, V = logprobs.shape
    L = beam_seq.shape[2]
    layers, _, _, H = state.shape
    RB = 32
    t_arr = jnp.asarray(t, jnp.int32).reshape(1, 1)
    xt = jnp.transpose(logprobs, (1, 0, 2))

    import functools
    scan_body = functools.partial(_scan_kernel, rows=RB, beam=beam, V=V)
    vals_t, idx_t = pl.pallas_call(
        scan_body,
        grid=(beam, B // RB),
        in_specs=[pl.BlockSpec((1, RB, V), lambda r, g: (r, g, 0))],
        out_specs=[
            pl.BlockSpec((1, RB, beam), lambda r, g: (r, g, 0)),
            pl.BlockSpec((1, RB, beam), lambda r, g: (r, g, 0)),
        ],
        out_shape=[
            jax.ShapeDtypeStruct((beam, B, beam), jnp.float32),
            jax.ShapeDtypeStruct((beam, B, beam), jnp.int32),
        ],
    )(xt)

    out = pl.pallas_call(
        _merge_kernel,
        grid=(1,),
        in_specs=[
            pl.BlockSpec((beam, B, beam), lambda g: (0, 0, 0)),
            pl.BlockSpec((beam, B, beam), lambda g: (0, 0, 0)),
            pl.BlockSpec((B, beam), lambda g: (0, 0)),
            pl.BlockSpec((beam, L, B), lambda g: (0, 0, 0)),
            pl.BlockSpec((beam, L, B), lambda g: (0, 0, 0)),
            pl.BlockSpec((layers, beam, B, H), lambda g: (0, 0, 0, 0)),
            pl.BlockSpec((1, 1), lambda g: (0, 0)),
        ],
        out_specs=[
            pl.BlockSpec((beam, L, B), lambda g: (0, 0, 0)),
            pl.BlockSpec((beam, L, B), lambda g: (0, 0, 0)),
            pl.BlockSpec((beam, B), lambda g: (0, 0)),
            pl.BlockSpec((layers, beam, B, H), lambda g: (0, 0, 0, 0)),
        ],
        out_shape=[
            jax.ShapeDtypeStruct((beam, L, B), jnp.int32),
            jax.ShapeDtypeStruct((beam, L, B), jnp.float32),
            jax.ShapeDtypeStruct((beam, B), jnp.float32),
            jax.ShapeDtypeStruct((layers, beam, B, H), jnp.float32),
        ],
    )(vals_t, idx_t, beam_logprobs_sum,
      jnp.transpose(beam_seq, (1, 2, 0)),
      jnp.transpose(beam_seq_logprobs, (1, 2, 0)),
      jnp.transpose(state, (0, 2, 1, 3)), t_arr)
    new_seq_t, new_seqlp_t, sums_t, new_state_t = out
    return (jnp.transpose(new_seq_t, (2, 0, 1)),
            jnp.transpose(new_seqlp_t, (2, 0, 1)),
            sums_t.T,
            jnp.transpose(new_state_t, (0, 2, 1, 3)))
